# baseline (speedup 1.0000x reference)
import functools
import math

import jax
import jax.numpy as jnp
from jax import lax
from jax.experimental import pallas as pl
from jax.experimental.pallas import tpu as pltpu
from jax.experimental.pallas import tpu_sc as plsc

_N = 2048
_V = 100000
_PAD = 0
_SMOOTH = 0.1
_CONF = 1.0 - _SMOOTH
_SV = _SMOOTH / (_V - 2)
_C0 = (_V - 2) * _SV * math.log(_SV) + _CONF * math.log(_CONF)
_NEG = float("-inf")
_BIG = 2 ** 30

_NW = 32
_RPW = _N // _NW
_GPW = _RPW // 8
_NCH = 32
_CW = 3200
_LASTW = 800
_LASTV = _LASTW // 16


def _group_chunk(buf, states, c32, nv):
    vb = c32 * (_CW // 16)

    def body(jj, st):
        out = []
        vecidx = vb + jj
        for r8 in range(8):
            acc, m, i = st[r8]
            v = buf[pl.ds(r8 * _CW + jj * 16, 16)]
            acc = acc + v
            upd = v > m
            m = jnp.maximum(m, v)
            i = jnp.where(upd, vecidx, i)
            out.append((acc, m, i))
        return tuple(out)

    return lax.fori_loop(0, nv, body, states)


def _sc_main_body(x_hbm, s_hbm, m_hbm, i_hbm,
                  buf0, buf1, s_st, m_st, i_st, sem0, sem1):
    wid = lax.axis_index("s") * 2 + lax.axis_index("c")
    r0 = wid * _RPW
    bufs = (buf0, buf1)
    sems = (sem0, sem1)

    def issue(grp, c32, b):
        w = _CW if c32 < _NCH - 1 else _LASTW
        for r8 in range(8):
            row = r0 + grp * 8 + r8
            pltpu.async_copy(
                x_hbm.at[pl.ds(row * _V + c32 * _CW, w)],
                bufs[b].at[pl.ds(r8 * _CW, w)], sems[b])

    def wait(c32, b):
        w = _CW if c32 < _NCH - 1 else _LASTW
        for r8 in range(8):
            pltpu.make_async_copy(
                x_hbm.at[pl.ds(0, w)],
                bufs[b].at[pl.ds(r8 * _CW, w)], sems[b]).wait()

    issue(0, 0, 0)

    def group_body(g, _):
        st = tuple((jnp.zeros((16,), jnp.float32),
                    jnp.full((16,), _NEG, jnp.float32),
                    jnp.zeros((16,), jnp.int32)) for _ in range(8))
        for c32 in range(_NCH):
            b = c32 % 2
            wait(c32, b)
            if c32 < _NCH - 1:
                issue(g, c32 + 1, 1 - b)
            else:
                issue(jnp.minimum(g + 1, _GPW - 1), 0, 1 - b)
            nv = _CW // 16 if c32 < _NCH - 1 else _LASTV
            st = _group_chunk(bufs[b], st, c32, nv)
        for r8 in range(8):
            acc, m, i = st[r8]
            base = (g * 8 + r8) * 16
            s_st[pl.ds(base, 16)] = acc
            m_st[pl.ds(base, 16)] = m
            i_st[pl.ds(base, 16)] = i
        return 0

    lax.fori_loop(0, _GPW, group_body, 0)
    wait(0, 0)
    pltpu.sync_copy(s_st, s_hbm.at[pl.ds(r0 * 16, _RPW * 16)])
    pltpu.sync_copy(m_st, m_hbm.at[pl.ds(r0 * 16, _RPW * 16)])
    pltpu.sync_copy(i_st, i_hbm.at[pl.ds(r0 * 16, _RPW * 16)])


@functools.cache
def _sc_main():
    return pl.kernel(
        _sc_main_body,
        out_type=[jax.ShapeDtypeStruct((_N * 16,), jnp.float32),
                  jax.ShapeDtypeStruct((_N * 16,), jnp.float32),
                  jax.ShapeDtypeStruct((_N * 16,), jnp.int32)],
        mesh=plsc.VectorSubcoreMesh(core_axis_name="c",
                                    subcore_axis_name="s"),
        scratch_types=[
            pltpu.VMEM((8 * _CW,), jnp.float32),
            pltpu.VMEM((8 * _CW,), jnp.float32),
            pltpu.VMEM((_RPW * 16,), jnp.float32),
            pltpu.VMEM((_RPW * 16,), jnp.float32),
            pltpu.VMEM((_RPW * 16,), jnp.int32),
            pltpu.SemaphoreType.DMA,
            pltpu.SemaphoreType.DMA,
        ],
    )


_CHUNKS = _RPW // 16


def _sc_gather_body(flat_hbm, tgt_hbm, xt_hbm, x0_hbm,
                    tgt_v, idxt_v, idx0_v, xt_v, x0_v, sem):
    wid = lax.axis_index("s") * 2 + lax.axis_index("c")
    base = wid * _RPW
    pltpu.sync_copy(tgt_hbm.at[pl.ds(base, _RPW)], tgt_v)
    iota = lax.iota(jnp.int32, 16)
    for k in range(_CHUNKS):
        rows = iota + (base + k * 16)
        t16 = tgt_v[pl.ds(k * 16, 16)]
        idx0_v[pl.ds(k * 16, 16)] = rows * _V
        idxt_v[pl.ds(k * 16, 16)] = rows * _V + t16
    pltpu.async_copy(flat_hbm.at[idxt_v], xt_v, sem).wait()
    pltpu.async_copy(flat_hbm.at[idx0_v], x0_v, sem).wait()
    pltpu.sync_copy(xt_v, xt_hbm.at[pl.ds(base, _RPW)])
    pltpu.sync_copy(x0_v, x0_hbm.at[pl.ds(base, _RPW)])


@functools.cache
def _sc_gather():
    return pl.kernel(
        _sc_gather_body,
        out_type=[jax.ShapeDtypeStruct((_N,), jnp.float32),
                  jax.ShapeDtypeStruct((_N,), jnp.float32)],
        mesh=plsc.VectorSubcoreMesh(core_axis_name="c",
                                    subcore_axis_name="s"),
        scratch_types=[
            pltpu.VMEM((_RPW,), jnp.int32),
            pltpu.VMEM((_RPW,), jnp.int32),
            pltpu.VMEM((_RPW,), jnp.int32),
            pltpu.VMEM((_RPW,), jnp.float32),
            pltpu.VMEM((_RPW,), jnp.float32),
            pltpu.SemaphoreType.DMA,
        ],
    )


def _fin_body(s_ref, m_ref, i_ref, t_ref, xt_ref, x0_ref,
              loss_ref, cor_ref, np_ref):
    s16 = s_ref[...]
    m16 = m_ref[...]
    i16 = i_ref[...]
    lane = lax.broadcasted_iota(jnp.int32, (_N, 16), 1)
    rsum = jnp.sum(s16, axis=1, keepdims=True)
    rmax = jnp.max(m16, axis=1, keepdims=True)
    cols = i16 * 16 + lane
    first = jnp.min(jnp.where(m16 == rmax, cols, _BIG), axis=1,
                    keepdims=True)
    t = t_ref[...]
    xt = xt_ref[...]
    x0 = x0_ref[...]
    nonpad = t != _PAD
    lrows = jnp.where(nonpad,
                      _C0 - _SV * (rsum - x0 - xt) - _CONF * xt, 0.0)
    loss_ref[0, 0] = jnp.sum(lrows)
    cor_ref[0, 0] = jnp.sum(jnp.where(nonpad & (first == t), 1, 0))
    np_ref[0, 0] = jnp.sum(nonpad.astype(jnp.int32))


def _finalize(s, m, i, t, xt, x0, interpret=False):
    return pl.pallas_call(
        _fin_body,
        out_specs=[
            pl.BlockSpec(memory_space=pltpu.SMEM),
            pl.BlockSpec(memory_space=pltpu.SMEM),
            pl.BlockSpec(memory_space=pltpu.SMEM),
        ],
        out_shape=[
            jax.ShapeDtypeStruct((1, 1), jnp.float32),
            jax.ShapeDtypeStruct((1, 1), jnp.int32),
            jax.ShapeDtypeStruct((1, 1), jnp.int32),
        ],
        interpret=interpret,
    )(s.reshape(_N, 16), m.reshape(_N, 16), i.reshape(_N, 16),
      t.reshape(_N, 1), xt.reshape(_N, 1), x0.reshape(_N, 1))


def kernel(output, target):
    target = target.astype(jnp.int32)
    flat = output.reshape(_N * _V)
    s, m, i = _sc_main()(flat)
    xt, x0 = _sc_gather()(flat, target)
    loss, cor, npd = _finalize(s, m, i, target, xt, x0)
    return loss[0, 0], cor[0, 0], npd[0, 0]
